# baseline (speedup 1.0000x reference)
import jax
import jax.numpy as jnp
from jax import lax
from jax.experimental import pallas as pl
from jax.experimental.pallas import tpu as pltpu, tpu_sc as plsc

_NC, _NS, _LANES = 2, 16, 16
_CH = 4
_NB = 16
_AHEAD = 12


def _sc_body(x_hbm, pos_hbm, out_hbm,
             xb0, xb1, xb2, xb3, xb4, xb5, xb6, xb7,
             xb8, xb9, xb10, xb11, xb12, xb13, xb14, xb15, pb0, pb1,
             xsem, osem, psem):
    BL, D = x_hbm.shape
    L = pos_hbm.shape[0]
    wid = lax.axis_index("s") * _NC + lax.axis_index("c")
    rows_per_w = L // (_NC * _NS)
    nchunks = rows_per_w // _CH
    nsteps = 4 * nchunks
    niters = nsteps // 16
    l0 = wid * rows_per_w
    xbuf = (xb0, xb1, xb2, xb3, xb4, xb5, xb6, xb7,
            xb8, xb9, xb10, xb11, xb12, xb13, xb14, xb15)
    pbuf = (pb0, pb1)

    def x_rows(g):
        return (g % 4) * L + l0 + (g // 4) * _CH

    def pos_rows(c):
        return l0 + c * _CH

    def load_x(g, buf):
        pltpu.async_copy(x_hbm.at[pl.ds(x_rows(g), _CH)], buf, xsem)

    def store_x(g, buf):
        pltpu.async_copy(buf, out_hbm.at[pl.ds(x_rows(g), _CH)], osem)

    def load_pos(c, buf):
        pltpu.async_copy(pos_hbm.at[pl.ds(pos_rows(c), _CH)], buf, psem)

    def wait_x(g, buf):
        pltpu.make_async_copy(x_hbm.at[pl.ds(x_rows(g), _CH)], buf, xsem).wait()

    def wait_store(g, buf):
        pltpu.make_async_copy(buf, out_hbm.at[pl.ds(x_rows(g), _CH)], osem).wait()

    def wait_pos(c, buf):
        pltpu.make_async_copy(
            pos_hbm.at[pl.ds(pos_rows(c), _CH)], buf, psem
        ).wait()

    for g in range(_AHEAD):
        load_x(g, xbuf[g % _NB])
    load_pos(0, pbuf[0])

    def iteration(i, _):
        for s in range(16):
            g = 16 * i + s
            cs = s // 4
            if s % 4 == 0 and cs < 3:
                load_pos(4 * i + cs + 1, pbuf[(cs + 1) % 2])
            if s == 12:
                @pl.when(i < niters - 1)
                def _():
                    load_pos(4 * i + 4, pbuf[0])

            tbuf = xbuf[(s + _AHEAD) % _NB]
            if s < _NB - _AHEAD:
                @pl.when(i > 0)
                def _():
                    wait_store(g + _AHEAD - _NB, tbuf)
            else:
                wait_store(g + _AHEAD - _NB, tbuf)
            if 16 * (niters - 1) + s + _AHEAD < nsteps:
                load_x(g + _AHEAD, tbuf)
            else:
                @pl.when(i < niters - 1)
                def _():
                    load_x(g + _AHEAD, tbuf)

            if s % 4 == 0:
                wait_pos(4 * i + cs, pbuf[cs % 2])
            wait_x(g, xbuf[s % _NB])

            xv = xbuf[s % _NB]
            pv = pbuf[cs % 2]

            def row(r, _):
                for c in range(D // _LANES):
                    off = c * _LANES
                    plsc.addupdate(
                        xv.at[r, pl.ds(off, _LANES)],
                        pv[r, pl.ds(off, _LANES)],
                    )
                return 0

            lax.fori_loop(0, _CH, row, 0)
            store_x(g, xv)
        return 0

    lax.fori_loop(0, niters, iteration, 0)
    for g in range(nsteps - (_NB - _AHEAD), nsteps):
        wait_store(g, xbuf[g % _NB])


def _sc_kernel(x_bld, pos_table):
    B, L, D = x_bld.shape
    x2 = x_bld.reshape(B * L, D)
    mesh = plsc.VectorSubcoreMesh(
        core_axis_name="c", subcore_axis_name="s",
        num_cores=_NC, num_subcores=_NS,
    )
    out = pl.kernel(
        _sc_body,
        out_type=jax.ShapeDtypeStruct((B * L, D), jnp.float32),
        mesh=mesh,
        scratch_types=(
            [pltpu.VMEM((_CH, D), jnp.float32)] * (_NB + 2)
            + [pltpu.SemaphoreType.DMA] * 3
        ),
    )(x2, pos_table)
    return out.reshape(B, L, D)


_TL = 2048


def _tc_body(x_ref, p_ref, o_ref):
    o_ref[...] = x_ref[...] + p_ref[...]


def _tc_kernel(x_bld, pos_table):
    B, L, D = x_bld.shape
    return pl.pallas_call(
        _tc_body,
        grid=(L // _TL, B),
        in_specs=[
            pl.BlockSpec((1, _TL, D), lambda l, b: (b, l, 0)),
            pl.BlockSpec((_TL, D), lambda l, b: (l, 0)),
        ],
        out_specs=pl.BlockSpec((1, _TL, D), lambda l, b: (b, l, 0)),
        out_shape=jax.ShapeDtypeStruct(x_bld.shape, x_bld.dtype),
    )(x_bld, pos_table)


def kernel(x_bld, pos_table):
    return _sc_kernel(x_bld, pos_table)


# speedup vs baseline: 1.0419x; 1.0419x over previous
import jax
import jax.numpy as jnp
from jax import lax
from jax.experimental import pallas as pl
from jax.experimental.pallas import tpu as pltpu, tpu_sc as plsc

_NC, _NS, _LANES = 2, 16, 16
_CH = 8
_NB = 8
_AHEAD = 7


def _sc_body(x_hbm, pos_hbm, out_hbm,
             xb0, xb1, xb2, xb3, xb4, xb5, xb6, xb7,
             pb0, pb1, xsem, osem, psem):
    BL, D = x_hbm.shape
    L = pos_hbm.shape[0]
    wid = lax.axis_index("s") * _NC + lax.axis_index("c")
    rows_per_w = L // (_NC * _NS)
    nchunks = rows_per_w // _CH
    nsteps = 4 * nchunks
    niters = nsteps // 16
    l0 = wid * rows_per_w
    xbuf = (xb0, xb1, xb2, xb3, xb4, xb5, xb6, xb7)
    pbuf = (pb0, pb1)

    def x_rows(g):
        return (g % 4) * L + l0 + (g // 4) * _CH

    def pos_rows(c):
        return l0 + c * _CH

    def load_x(g, buf):
        pltpu.async_copy(x_hbm.at[pl.ds(x_rows(g), _CH)], buf, xsem)

    def store_x(g, buf):
        pltpu.async_copy(buf, out_hbm.at[pl.ds(x_rows(g), _CH)], osem)

    def load_pos(c, buf):
        pltpu.async_copy(pos_hbm.at[pl.ds(pos_rows(c), _CH)], buf, psem)

    def wait_x(g, buf):
        pltpu.make_async_copy(x_hbm.at[pl.ds(x_rows(g), _CH)], buf, xsem).wait()

    def wait_store(g, buf):
        pltpu.make_async_copy(buf, out_hbm.at[pl.ds(x_rows(g), _CH)], osem).wait()

    def wait_pos(c, buf):
        pltpu.make_async_copy(
            pos_hbm.at[pl.ds(pos_rows(c), _CH)], buf, psem
        ).wait()

    for g in range(_AHEAD):
        load_x(g, xbuf[g % _NB])
    load_pos(0, pbuf[0])

    def iteration(i, _):
        for s in range(16):
            g = 16 * i + s
            cs = s // 4
            if s % 4 == 0 and cs < 3:
                load_pos(4 * i + cs + 1, pbuf[(cs + 1) % 2])
            if s == 12:
                @pl.when(i < niters - 1)
                def _():
                    load_pos(4 * i + 4, pbuf[0])

            tbuf = xbuf[(s + _AHEAD) % _NB]
            if s < _NB - _AHEAD:
                @pl.when(i > 0)
                def _():
                    wait_store(g + _AHEAD - _NB, tbuf)
            else:
                wait_store(g + _AHEAD - _NB, tbuf)
            if 16 * (niters - 1) + s + _AHEAD < nsteps:
                load_x(g + _AHEAD, tbuf)
            else:
                @pl.when(i < niters - 1)
                def _():
                    load_x(g + _AHEAD, tbuf)

            if s % 4 == 0:
                wait_pos(4 * i + cs, pbuf[cs % 2])
            wait_x(g, xbuf[s % _NB])

            xv = xbuf[s % _NB]
            pv = pbuf[cs % 2]

            def row(r, _):
                for c in range(D // _LANES):
                    off = c * _LANES
                    plsc.addupdate(
                        xv.at[r, pl.ds(off, _LANES)],
                        pv[r, pl.ds(off, _LANES)],
                    )
                return 0

            lax.fori_loop(0, _CH, row, 0)
            store_x(g, xv)
        return 0

    lax.fori_loop(0, niters, iteration, 0)
    for g in range(nsteps - (_NB - _AHEAD), nsteps):
        wait_store(g, xbuf[g % _NB])


def _sc_kernel(x_bld, pos_table):
    B, L, D = x_bld.shape
    x2 = x_bld.reshape(B * L, D)
    mesh = plsc.VectorSubcoreMesh(
        core_axis_name="c", subcore_axis_name="s",
        num_cores=_NC, num_subcores=_NS,
    )
    out = pl.kernel(
        _sc_body,
        out_type=jax.ShapeDtypeStruct((B * L, D), jnp.float32),
        mesh=mesh,
        scratch_types=(
            [pltpu.VMEM((_CH, D), jnp.float32)] * (_NB + 2)
            + [pltpu.SemaphoreType.DMA] * 3
        ),
    )(x2, pos_table)
    return out.reshape(B, L, D)


_TL = 2048


def _tc_body(x_ref, p_ref, o_ref):
    o_ref[...] = x_ref[...] + p_ref[...]


def _tc_kernel(x_bld, pos_table):
    B, L, D = x_bld.shape
    return pl.pallas_call(
        _tc_body,
        grid=(L // _TL, B),
        in_specs=[
            pl.BlockSpec((1, _TL, D), lambda l, b: (b, l, 0)),
            pl.BlockSpec((_TL, D), lambda l, b: (l, 0)),
        ],
        out_specs=pl.BlockSpec((1, _TL, D), lambda l, b: (b, l, 0)),
        out_shape=jax.ShapeDtypeStruct(x_bld.shape, x_bld.dtype),
    )(x_bld, pos_table)


def kernel(x_bld, pos_table):
    return _sc_kernel(x_bld, pos_table)


# speedup vs baseline: 1.1801x; 1.1326x over previous
import jax
import jax.numpy as jnp
from jax import lax
from jax.experimental import pallas as pl
from jax.experimental.pallas import tpu as pltpu, tpu_sc as plsc

_NC, _NS, _LANES = 2, 16, 16
_CH = 8
_NB = 8
_AHEAD = 6


def _sc_body(x_hbm, pos_hbm, out_hbm,
             xb0, xb1, xb2, xb3, xb4, xb5, xb6, xb7,
             pb0, pb1, xsem, osem, psem):
    BL, D = x_hbm.shape
    L = pos_hbm.shape[0]
    wid = lax.axis_index("s") * _NC + lax.axis_index("c")
    rows_per_w = L // (_NC * _NS)
    nchunks = rows_per_w // _CH
    nsteps = 4 * nchunks
    niters = nsteps // 16
    l0 = wid * rows_per_w
    xbuf = (xb0, xb1, xb2, xb3, xb4, xb5, xb6, xb7)
    pbuf = (pb0, pb1)

    def x_rows(g):
        return (g % 4) * L + l0 + (g // 4) * _CH

    def pos_rows(c):
        return l0 + c * _CH

    def load_x(g, buf):
        pltpu.async_copy(x_hbm.at[pl.ds(x_rows(g), _CH)], buf, xsem)

    def store_x(g, buf):
        pltpu.async_copy(buf, out_hbm.at[pl.ds(x_rows(g), _CH)], osem)

    def load_pos(c, buf):
        pltpu.async_copy(pos_hbm.at[pl.ds(pos_rows(c), _CH)], buf, psem)

    def wait_x(g, buf):
        pltpu.make_async_copy(x_hbm.at[pl.ds(x_rows(g), _CH)], buf, xsem).wait()

    def wait_store(g, buf):
        pltpu.make_async_copy(buf, out_hbm.at[pl.ds(x_rows(g), _CH)], osem).wait()

    def wait_pos(c, buf):
        pltpu.make_async_copy(
            pos_hbm.at[pl.ds(pos_rows(c), _CH)], buf, psem
        ).wait()

    for g in range(_AHEAD):
        load_x(g, xbuf[g % _NB])
    load_pos(0, pbuf[0])

    def iteration(i, _):
        for s in range(16):
            g = 16 * i + s
            cs = s // 4
            if s % 4 == 0 and cs < 3:
                load_pos(4 * i + cs + 1, pbuf[(cs + 1) % 2])
            if s == 12:
                @pl.when(i < niters - 1)
                def _():
                    load_pos(4 * i + 4, pbuf[0])

            tbuf = xbuf[(s + _AHEAD) % _NB]
            if s < _NB - _AHEAD:
                @pl.when(i > 0)
                def _():
                    wait_store(g + _AHEAD - _NB, tbuf)
            else:
                wait_store(g + _AHEAD - _NB, tbuf)
            if 16 * (niters - 1) + s + _AHEAD < nsteps:
                load_x(g + _AHEAD, tbuf)
            else:
                @pl.when(i < niters - 1)
                def _():
                    load_x(g + _AHEAD, tbuf)

            if s % 4 == 0:
                wait_pos(4 * i + cs, pbuf[cs % 2])
            wait_x(g, xbuf[s % _NB])

            xv = xbuf[s % _NB]
            pv = pbuf[cs % 2]

            def row(r, _):
                for c in range(D // _LANES):
                    off = c * _LANES
                    plsc.addupdate(
                        xv.at[r, pl.ds(off, _LANES)],
                        pv[r, pl.ds(off, _LANES)],
                    )
                return 0

            lax.fori_loop(0, _CH, row, 0)
            store_x(g, xv)
        return 0

    lax.fori_loop(0, niters, iteration, 0)
    for g in range(nsteps - (_NB - _AHEAD), nsteps):
        wait_store(g, xbuf[g % _NB])


def _sc_kernel(x_bld, pos_table):
    B, L, D = x_bld.shape
    x2 = x_bld.reshape(B * L, D)
    mesh = plsc.VectorSubcoreMesh(
        core_axis_name="c", subcore_axis_name="s",
        num_cores=_NC, num_subcores=_NS,
    )
    out = pl.kernel(
        _sc_body,
        out_type=jax.ShapeDtypeStruct((B * L, D), jnp.float32),
        mesh=mesh,
        scratch_types=(
            [pltpu.VMEM((_CH, D), jnp.float32)] * (_NB + 2)
            + [pltpu.SemaphoreType.DMA] * 3
        ),
    )(x2, pos_table)
    return out.reshape(B, L, D)


_TL = 2048


def _tc_body(x_ref, p_ref, o_ref):
    o_ref[...] = x_ref[...] + p_ref[...]


def _tc_kernel(x_bld, pos_table):
    B, L, D = x_bld.shape
    return pl.pallas_call(
        _tc_body,
        grid=(L // _TL, B),
        in_specs=[
            pl.BlockSpec((1, _TL, D), lambda l, b: (b, l, 0)),
            pl.BlockSpec((_TL, D), lambda l, b: (l, 0)),
        ],
        out_specs=pl.BlockSpec((1, _TL, D), lambda l, b: (b, l, 0)),
        out_shape=jax.ShapeDtypeStruct(x_bld.shape, x_bld.dtype),
    )(x_bld, pos_table)


def kernel(x_bld, pos_table):
    return _sc_kernel(x_bld, pos_table)


# speedup vs baseline: 1.2576x; 1.0656x over previous
import jax
import jax.numpy as jnp
from jax import lax
from jax.experimental import pallas as pl
from jax.experimental.pallas import tpu as pltpu, tpu_sc as plsc

_NC, _NS, _LANES = 2, 16, 16
_CH = 8
_NB = 8
_AHEAD = 6


def _sc_body(x_hbm, pos_hbm, out_hbm,
             xb0, xb1, xb2, xb3, xb4, xb5, xb6, xb7,
             pb0, pb1, xsem, osem, psem):
    BL, D = x_hbm.shape
    L = pos_hbm.shape[0]
    wid = lax.axis_index("s") * _NC + lax.axis_index("c")
    rows_per_w = L // (_NC * _NS)
    nchunks = rows_per_w // _CH
    nsteps = 4 * nchunks
    niters = nsteps // 8
    l0 = wid * rows_per_w
    xbuf = (xb0, xb1, xb2, xb3, xb4, xb5, xb6, xb7)
    pbuf = (pb0, pb1)

    def x_rows(g):
        return (g % 4) * L + l0 + (g // 4) * _CH

    def pos_rows(c):
        return l0 + c * _CH

    def load_x(g, buf):
        pltpu.async_copy(x_hbm.at[pl.ds(x_rows(g), _CH)], buf, xsem)

    def store_x(g, buf):
        pltpu.async_copy(buf, out_hbm.at[pl.ds(x_rows(g), _CH)], osem)

    def load_pos(c, buf):
        pltpu.async_copy(pos_hbm.at[pl.ds(pos_rows(c), _CH)], buf, psem)

    def wait_x(g, buf):
        pltpu.make_async_copy(x_hbm.at[pl.ds(x_rows(g), _CH)], buf, xsem).wait()

    def wait_store(g, buf):
        pltpu.make_async_copy(buf, out_hbm.at[pl.ds(x_rows(g), _CH)], osem).wait()

    def wait_pos(c, buf):
        pltpu.make_async_copy(
            pos_hbm.at[pl.ds(pos_rows(c), _CH)], buf, psem
        ).wait()

    for g in range(_AHEAD):
        load_x(g, xbuf[g % _NB])
    load_pos(0, pbuf[0])

    def iteration(i, _):
        for s in range(8):
            g = 8 * i + s
            if s == 0:
                load_pos(2 * i + 1, pbuf[1])
            if s == 4:
                @pl.when(i < niters - 1)
                def _():
                    load_pos(2 * i + 2, pbuf[0])

            tbuf = xbuf[(s + _AHEAD) % _NB]
            if s < _NB - _AHEAD:
                @pl.when(i > 0)
                def _():
                    wait_store(g + _AHEAD - _NB, tbuf)
            else:
                wait_store(g + _AHEAD - _NB, tbuf)
            if 8 * (niters - 1) + s + _AHEAD < nsteps:
                load_x(g + _AHEAD, tbuf)
            else:
                @pl.when(i < niters - 1)
                def _():
                    load_x(g + _AHEAD, tbuf)

            if s == 0:
                wait_pos(2 * i, pbuf[0])
            if s == 4:
                wait_pos(2 * i + 1, pbuf[1])
            wait_x(g, xbuf[s % _NB])

            xv = xbuf[s % _NB]
            pv = pbuf[s // 4]

            def row(r, _):
                for c in range(D // _LANES):
                    off = c * _LANES
                    plsc.addupdate(
                        xv.at[r, pl.ds(off, _LANES)],
                        pv[r, pl.ds(off, _LANES)],
                    )
                return 0

            lax.fori_loop(0, _CH, row, 0)
            store_x(g, xv)
        return 0

    lax.fori_loop(0, niters, iteration, 0)
    for g in range(nsteps - (_NB - _AHEAD), nsteps):
        wait_store(g, xbuf[g % _NB])


def _sc_kernel(x_bld, pos_table):
    B, L, D = x_bld.shape
    x2 = x_bld.reshape(B * L, D)
    mesh = plsc.VectorSubcoreMesh(
        core_axis_name="c", subcore_axis_name="s",
        num_cores=_NC, num_subcores=_NS,
    )
    out = pl.kernel(
        _sc_body,
        out_type=jax.ShapeDtypeStruct((B * L, D), jnp.float32),
        mesh=mesh,
        scratch_types=(
            [pltpu.VMEM((_CH, D), jnp.float32)] * (_NB + 2)
            + [pltpu.SemaphoreType.DMA] * 3
        ),
    )(x2, pos_table)
    return out.reshape(B, L, D)


_TL = 2048


def _tc_body(x_ref, p_ref, o_ref):
    o_ref[...] = x_ref[...] + p_ref[...]


def _tc_kernel(x_bld, pos_table):
    B, L, D = x_bld.shape
    return pl.pallas_call(
        _tc_body,
        grid=(L // _TL, B),
        in_specs=[
            pl.BlockSpec((1, _TL, D), lambda l, b: (b, l, 0)),
            pl.BlockSpec((_TL, D), lambda l, b: (l, 0)),
        ],
        out_specs=pl.BlockSpec((1, _TL, D), lambda l, b: (b, l, 0)),
        out_shape=jax.ShapeDtypeStruct(x_bld.shape, x_bld.dtype),
    )(x_bld, pos_table)


def kernel(x_bld, pos_table):
    return _sc_kernel(x_bld, pos_table)
